# baseline (speedup 1.0000x reference)
import functools

import jax
import jax.numpy as jnp
import numpy as np
from jax import lax
from jax.experimental import pallas as pl
from jax.experimental.pallas import tpu as pltpu
from jax.experimental.pallas import tpu_sc as plsc

_NUM_CORES = 2
_NUM_SUBCORES = 16
_NW = _NUM_CORES * _NUM_SUBCORES
_GATHER = 128
_G = 2
_CHUNK = _G * _GATHER
_NBUF = 3
_LANES = 16


def _make_kernel(B, V, D):
    nfull = B // _CHUNK
    tail = B - nfull * _CHUNK
    kmax = -(-nfull // _NW)
    n_stage8 = V // 8
    v_rem = V - n_stage8 * 8
    scale = float(np.float32(1.0) / np.sqrt(np.float32(V)))

    mesh = plsc.VectorSubcoreMesh(core_axis_name="c", subcore_axis_name="s")

    @functools.partial(
        pl.kernel,
        mesh=mesh,
        out_type=jax.ShapeDtypeStruct((B, D), jnp.float32),
        scratch_types=[
            pltpu.VMEM((_NBUF, _G, _GATHER), jnp.int32),
            pltpu.VMEM((_NBUF, _CHUNK, D), jnp.float32),
            pltpu.VMEM((8, D), jnp.float32),
            pltpu.VMEM((max((B - (B // _CHUNK) * _CHUNK) % _GATHER, 8),),
                       jnp.int32),
            pltpu.VMEM_SHARED((V, D), jnp.float32),
            pltpu.SemaphoreType.DMA,
            pltpu.SemaphoreType.DMA,
            pltpu.SemaphoreType.DMA,
            pltpu.SemaphoreType.DMA,
            pltpu.SemaphoreType.DMA,
            pltpu.SemaphoreType.DMA,
            pltpu.SemaphoreType.DMA,
            pltpu.SemaphoreType.DMA,
            pltpu.SemaphoreType.DMA,
        ],
    )
    def k(emb_hbm, idx_hbm, out_hbm, idx_v, rows_v, tab_v, tidx_v, w_sp,
          gsem0, gsem1, gsem2, isem0, isem1, isem2, ssem0, ssem1, ssem2):
        gsem = (gsem0, gsem1, gsem2)
        isem = (isem0, isem1, isem2)
        ssem = (ssem0, ssem1, ssem2)
        s = lax.axis_index("s")
        wid = s * _NUM_CORES + lax.axis_index("c")

        def stage(r0, nr):
            pltpu.sync_copy(emb_hbm.at[pl.ds(r0, nr)], tab_v.at[pl.ds(0, nr)])
            for dr in range(nr):
                for j in range(D // _LANES):
                    col = pl.ds(j * _LANES, _LANES)
                    tab_v[dr, col] = tab_v[dr, col] * scale
            pltpu.sync_copy(tab_v.at[pl.ds(0, nr)], w_sp.at[pl.ds(r0, nr)])

        @pl.when(s < n_stage8)
        def _():
            stage(s * 8, 8)

        if v_rem:
            @pl.when(s == n_stage8)
            def _():
                stage(n_stage8 * 8, v_rem)

        plsc.subcore_barrier()

        def cid(k_):
            return wid + k_ * _NW

        def idx_descs(k_, b):
            return [pltpu.make_async_copy(
                idx_hbm.at[pl.ds(cid(k_) * _CHUNK + g * _GATHER, _GATHER)],
                idx_v.at[b, g], isem[b]) for g in range(_G)]

        def gather_descs(k_, b):
            return [pltpu.make_async_copy(
                w_sp.at[idx_v.at[b, g]],
                rows_v.at[b, pl.ds(g * _GATHER, _GATHER)],
                gsem[b]) for g in range(_G)]

        def store_desc(k_, b):
            return pltpu.make_async_copy(
                rows_v.at[b], out_hbm.at[pl.ds(cid(k_) * _CHUNK, _CHUNK)],
                ssem[b])

        def when_valid(k_, fn):
            @pl.when(cid(k_) < nfull)
            def _():
                fn()

        def step(k_, ph, drain=True):
            b1 = (ph + 1) % _NBUF
            b2 = (ph + 2) % _NBUF
            def starts(ds):
                return lambda: [d.start() for d in ds]

            def waits(ds):
                return lambda: [d.wait() for d in ds]

            if drain:
                when_valid(k_ + 1, lambda: store_desc(k_ - 2, b1).wait())
            when_valid(k_ + 1, waits(idx_descs(k_ + 1, b1)))
            when_valid(k_ + 1, starts(gather_descs(k_ + 1, b1)))
            when_valid(k_, waits(gather_descs(k_, ph)))
            when_valid(k_, lambda: store_desc(k_, ph).start())
            when_valid(k_ + 2, starts(idx_descs(k_ + 2, b2)))

        when_valid(0, lambda: [d.start() for d in idx_descs(0, 0)])
        when_valid(1, lambda: [d.start() for d in idx_descs(1, 1)])
        when_valid(0, lambda: [d.wait() for d in idx_descs(0, 0)])
        when_valid(0, lambda: [d.start() for d in gather_descs(0, 0)])
        step(0, 0, drain=False)
        step(1, 1, drain=False)

        ntrip = (kmax - 2) // _NBUF

        @pl.loop(0, ntrip)
        def _(t):
            base_k = 2 + t * _NBUF
            for i in range(_NBUF):
                step(base_k + i, (2 + i) % _NBUF)

        for k_ in range(2 + ntrip * _NBUF, kmax):
            step(k_, k_ % _NBUF)

        for k_ in range(max(0, kmax - 4), kmax):
            @pl.when((cid(k_) < nfull) & (cid(k_ + 3) >= nfull))
            def _():
                store_desc(k_, k_ % _NBUF).wait()

        if tail:
            t_full = tail // _GATHER
            t_rem = tail - t_full * _GATHER

            def rem_gather():
                return pltpu.make_async_copy(
                    w_sp.at[tidx_v],
                    rows_v.at[0, pl.ds(t_full * _GATHER, t_rem)], gsem[0])

            @pl.when(wid == (nfull % _NW))
            def _():
                base = nfull * _CHUNK
                for g in range(t_full):
                    pltpu.sync_copy(
                        idx_hbm.at[pl.ds(base + g * _GATHER, _GATHER)],
                        idx_v.at[0, g])
                if t_rem:
                    pltpu.sync_copy(
                        idx_hbm.at[pl.ds(base + t_full * _GATHER, t_rem)],
                        tidx_v)
                for g in range(t_full):
                    pltpu.make_async_copy(
                        w_sp.at[idx_v.at[0, g]],
                        rows_v.at[0, pl.ds(g * _GATHER, _GATHER)],
                        gsem[0]).start()
                if t_rem:
                    rem_gather().start()
                for g in range(t_full):
                    pltpu.make_async_copy(
                        w_sp.at[idx_v.at[0, g]],
                        rows_v.at[0, pl.ds(g * _GATHER, _GATHER)],
                        gsem[0]).wait()
                if t_rem:
                    rem_gather().wait()
                pltpu.sync_copy(rows_v.at[0, pl.ds(0, tail)],
                                out_hbm.at[pl.ds(base, tail)])

    return k


def kernel(node_species, embeddings):
    V, D = embeddings.shape
    B = node_species.shape[0]
    idx = node_species.astype(jnp.int32)
    return _make_kernel(B, V, D)(embeddings, idx)


# speedup vs baseline: 1.0554x; 1.0554x over previous
import functools

import jax
import jax.numpy as jnp
import numpy as np
from jax import lax
from jax.experimental import pallas as pl
from jax.experimental.pallas import tpu as pltpu
from jax.experimental.pallas import tpu_sc as plsc

_NUM_CORES = 2
_NUM_SUBCORES = 16
_NW = _NUM_CORES * _NUM_SUBCORES
_CHUNK = 128
_NBUF = 3
_LANES = 16


def _make_kernel(B, V, D):
    nfull = B // _CHUNK
    tail = B - nfull * _CHUNK
    kmax = -(-nfull // _NW)
    n_stage8 = V // 8
    v_rem = V - n_stage8 * 8
    scale = float(np.float32(1.0) / np.sqrt(np.float32(V)))

    mesh = plsc.VectorSubcoreMesh(core_axis_name="c", subcore_axis_name="s")

    @functools.partial(
        pl.kernel,
        mesh=mesh,
        out_type=jax.ShapeDtypeStruct((B, D), jnp.float32),
        scratch_types=[
            pltpu.VMEM((_NBUF, _CHUNK), jnp.int32),
            pltpu.VMEM((_NBUF, _CHUNK, D), jnp.float32),
            pltpu.VMEM((8, D), jnp.float32),
            pltpu.VMEM((max(tail, 8),), jnp.int32),
            pltpu.VMEM_SHARED((V, D), jnp.float32),
            pltpu.SemaphoreType.DMA,
            pltpu.SemaphoreType.DMA,
            pltpu.SemaphoreType.DMA,
            pltpu.SemaphoreType.DMA,
            pltpu.SemaphoreType.DMA,
            pltpu.SemaphoreType.DMA,
            pltpu.SemaphoreType.DMA,
            pltpu.SemaphoreType.DMA,
            pltpu.SemaphoreType.DMA,
        ],
    )
    def k(emb_hbm, idx_hbm, out_hbm, idx_v, rows_v, tab_v, tidx_v, w_sp,
          gsem0, gsem1, gsem2, isem0, isem1, isem2, ssem0, ssem1, ssem2):
        gsem = (gsem0, gsem1, gsem2)
        isem = (isem0, isem1, isem2)
        ssem = (ssem0, ssem1, ssem2)
        s = lax.axis_index("s")
        wid = s * _NUM_CORES + lax.axis_index("c")

        def cid(k_):
            return wid + k_ * _NW

        def idx_desc(k_, b):
            return pltpu.make_async_copy(
                idx_hbm.at[pl.ds(cid(k_) * _CHUNK, _CHUNK)],
                idx_v.at[b], isem[b])

        def when_valid(k_, fn):
            @pl.when(cid(k_) < nfull)
            def _():
                fn()

        when_valid(0, lambda: idx_desc(0, 0).start())
        when_valid(1, lambda: idx_desc(1, 1).start())

        def stage(r0, nr):
            pltpu.sync_copy(emb_hbm.at[pl.ds(r0, nr)], tab_v.at[pl.ds(0, nr)])
            for dr in range(nr):
                for j in range(D // _LANES):
                    col = pl.ds(j * _LANES, _LANES)
                    tab_v[dr, col] = tab_v[dr, col] * scale
            pltpu.sync_copy(tab_v.at[pl.ds(0, nr)], w_sp.at[pl.ds(r0, nr)])

        @pl.when(s < n_stage8)
        def _():
            stage(s * 8, 8)

        if v_rem:
            @pl.when(s == n_stage8)
            def _():
                stage(n_stage8 * 8, v_rem)

        plsc.subcore_barrier()

        def gather_desc(k_, b):
            return pltpu.make_async_copy(
                w_sp.at[idx_v.at[b]], rows_v.at[b], gsem[b])

        def store_desc(k_, b):
            return pltpu.make_async_copy(
                rows_v.at[b], out_hbm.at[pl.ds(cid(k_) * _CHUNK, _CHUNK)],
                ssem[b])

        def step(k_, ph, drain=True):
            b1 = (ph + 1) % _NBUF
            b2 = (ph + 2) % _NBUF
            if drain:
                when_valid(k_ + 1, lambda: store_desc(k_ - 2, b1).wait())
            when_valid(k_ + 1, lambda: idx_desc(k_ + 1, b1).wait())
            when_valid(k_ + 1, lambda: gather_desc(k_ + 1, b1).start())
            when_valid(k_, lambda: gather_desc(k_, ph).wait())
            when_valid(k_, lambda: store_desc(k_, ph).start())
            when_valid(k_ + 2, lambda: idx_desc(k_ + 2, b2).start())

        when_valid(0, lambda: idx_desc(0, 0).wait())
        when_valid(0, lambda: gather_desc(0, 0).start())
        step(0, 0, drain=False)
        step(1, 1, drain=False)

        ntrip = (kmax - 2) // _NBUF

        @pl.loop(0, ntrip)
        def _(t):
            base_k = 2 + t * _NBUF
            for i in range(_NBUF):
                step(base_k + i, (2 + i) % _NBUF)

        for k_ in range(2 + ntrip * _NBUF, kmax):
            step(k_, k_ % _NBUF)

        for k_ in range(max(0, kmax - 4), kmax):
            @pl.when((cid(k_) < nfull) & (cid(k_ + 3) >= nfull))
            def _():
                store_desc(k_, k_ % _NBUF).wait()

        if tail:
            @pl.when(wid == (nfull % _NW))
            def _():
                base = nfull * _CHUNK
                pltpu.sync_copy(idx_hbm.at[pl.ds(base, tail)], tidx_v)
                pltpu.make_async_copy(
                    w_sp.at[tidx_v], rows_v.at[0, pl.ds(0, tail)],
                    gsem[0]).start()
                pltpu.make_async_copy(
                    w_sp.at[tidx_v], rows_v.at[0, pl.ds(0, tail)],
                    gsem[0]).wait()
                pltpu.sync_copy(rows_v.at[0, pl.ds(0, tail)],
                                out_hbm.at[pl.ds(base, tail)])

    return k


def kernel(node_species, embeddings):
    V, D = embeddings.shape
    B = node_species.shape[0]
    idx = node_species.astype(jnp.int32)
    return _make_kernel(B, V, D)(embeddings, idx)
